# baseline (speedup 1.0000x reference)
import functools

import jax
import jax.numpy as jnp
from jax import lax
from jax.experimental import pallas as pl
from jax.experimental.pallas import tpu as pltpu
from jax.experimental.pallas import tpu_sc as plsc

DIM = 256
KSIZE = 8192
M_TILE = 256
K_WIN = 2048


def _bf16_round(v):
    return v.astype(jnp.bfloat16).astype(jnp.float32)


def _dist_argmin_body(x_ref, cbt_ref, xsq_ref, esq_ref, idx_ref, loss_ref,
                      acc_ref):
    i = pl.program_id(0)

    @pl.when(i == 0)
    def _():
        acc_ref[0] = 0.0

    x = x_ref[...]
    xsq = xsq_ref[0, 0, :][:, None]

    iota_l = lax.broadcasted_iota(jnp.int32, (M_TILE, K_WIN), 1)
    acc_v = acc_i = fresh_v = None
    for w in range(KSIZE // K_WIN):
        off = w * K_WIN
        cbt_w = cbt_ref[:, pl.ds(off, K_WIN)]
        dots = lax.dot_general(x, cbt_w, (((1,), (0,)), ((), ())),
                               preferred_element_type=jnp.float32)
        esq_w = esq_ref[:, pl.ds(off, K_WIN)]
        dist = (xsq + dots) + esq_w
        wv = jnp.min(dist, axis=1)
        wi = jnp.min(jnp.where(dist == wv[:, None], iota_l, KSIZE),
                     axis=1) + off
        if acc_v is None:
            acc_v, acc_i, fresh_v = _bf16_round(wv), wi, wv
        else:
            upd = wv < acc_v
            acc_i = jnp.where(upd, wi, acc_i)
            fresh_v = jnp.where(upd, wv, fresh_v)
            acc_v = jnp.where(upd, _bf16_round(wv), acc_v)

    idx_ref[...] = acc_i
    acc_ref[0] += jnp.sum(fresh_v)

    @pl.when(i == pl.num_programs(0) - 1)
    def _():
        loss_ref[0, 0] = acc_ref[0] * (1.0 / (KSIZE * DIM))


def _dist_argmin(flat, cbt, xsq3, esq2):
    m = flat.shape[0]
    n_blocks = m // M_TILE
    return pl.pallas_call(
        _dist_argmin_body,
        grid=(n_blocks,),
        in_specs=[
            pl.BlockSpec((M_TILE, DIM), lambda i: (i, 0)),
            pl.BlockSpec((DIM, KSIZE), lambda i: (0, 0)),
            pl.BlockSpec((1, 1, M_TILE), lambda i: (i, 0, 0)),
            pl.BlockSpec((1, KSIZE), lambda i: (0, 0)),
        ],
        out_specs=[
            pl.BlockSpec((M_TILE,), lambda i: (i,)),
            pl.BlockSpec(memory_space=pltpu.SMEM),
        ],
        out_shape=[
            jax.ShapeDtypeStruct((m,), jnp.int32),
            jax.ShapeDtypeStruct((1, 1), jnp.float32),
        ],
        scratch_shapes=[pltpu.SMEM((1,), jnp.float32)],
    )(flat, cbt, xsq3, esq2)


def _sc_gather(codebook, idx_flat):
    nc, ns = 2, 16
    nw = nc * ns
    b = idx_flat.shape[0]
    b_per_w = b // nw
    mesh = plsc.VectorSubcoreMesh(core_axis_name="c", subcore_axis_name="s")

    @functools.partial(
        pl.kernel,
        mesh=mesh,
        out_type=jax.ShapeDtypeStruct((b, DIM), jnp.float32),
        scratch_types=[
            pltpu.VMEM((b_per_w,), jnp.int32),
            pltpu.VMEM((b_per_w, DIM), jnp.float32),
            pltpu.SemaphoreType.DMA,
        ],
    )
    def gather_kernel(cb_hbm, idx_hbm, out_hbm, idx_v, rows_v, sem):
        wid = lax.axis_index("s") * nc + lax.axis_index("c")
        base = wid * b_per_w
        pltpu.sync_copy(idx_hbm.at[pl.ds(base, b_per_w)], idx_v)
        pltpu.async_copy(cb_hbm.at[idx_v], rows_v, sem).wait()
        pltpu.sync_copy(rows_v, out_hbm.at[pl.ds(base, b_per_w)])

    return gather_kernel(codebook, idx_flat)


def kernel(x, codebook):
    b, n, d = x.shape
    flat = x.reshape(-1, d)
    m = flat.shape[0]
    x_sq = jnp.sum(flat * flat, axis=-1, keepdims=True)
    e_sq = jnp.sum(codebook * codebook, axis=-1)
    cbt = -2.0 * codebook.T
    xsq3 = x_sq.reshape(m // M_TILE, 1, M_TILE)
    esq2 = e_sq.reshape(1, KSIZE)

    idx_flat, loss = _dist_argmin(flat, cbt, xsq3, esq2)

    quantized = _sc_gather(codebook, idx_flat)

    return quantized.reshape(b, n, d), idx_flat.reshape(b, n), loss[0, 0]


# speedup vs baseline: 1.0327x; 1.0327x over previous
import functools

import jax
import jax.numpy as jnp
from jax import lax
from jax.experimental import pallas as pl
from jax.experimental.pallas import tpu as pltpu
from jax.experimental.pallas import tpu_sc as plsc

DIM = 256
KSIZE = 8192
M_TILE = 256
K_WIN = 2048


def _bf16_round(v):
    return v.astype(jnp.bfloat16).astype(jnp.float32)


def _dist_argmin_body(x_ref, cbt_ref, xsq_ref, esq_ref, idx_ref, loss_ref,
                      acc_ref):
    i = pl.program_id(0)

    @pl.when(i == 0)
    def _():
        acc_ref[0] = 0.0

    x = x_ref[...]
    xsq = xsq_ref[0, 0, :][:, None]

    iota_l = lax.broadcasted_iota(jnp.int32, (M_TILE, K_WIN), 1)
    acc_v = acc_i = fresh_v = None
    for w in range(KSIZE // K_WIN):
        off = w * K_WIN
        cb_w = cbt_ref[pl.ds(off, K_WIN), :]
        dots = lax.dot_general(x, cb_w, (((1,), (1,)), ((), ())),
                               preferred_element_type=jnp.float32)
        esq_w = esq_ref[:, pl.ds(off, K_WIN)]
        dist = xsq - 2.0 * dots + esq_w
        wv = jnp.min(dist, axis=1)
        wi = jnp.min(jnp.where(dist == wv[:, None], iota_l, KSIZE),
                     axis=1) + off
        if acc_v is None:
            acc_v, acc_i, fresh_v = _bf16_round(wv), wi, wv
        else:
            upd = wv < acc_v
            acc_i = jnp.where(upd, wi, acc_i)
            fresh_v = jnp.where(upd, wv, fresh_v)
            acc_v = jnp.where(upd, _bf16_round(wv), acc_v)

    idx_ref[...] = acc_i
    acc_ref[0] += jnp.sum(fresh_v)

    @pl.when(i == pl.num_programs(0) - 1)
    def _():
        loss_ref[0, 0] = acc_ref[0] * (1.0 / (KSIZE * DIM))


def _dist_argmin(flat, cbt, xsq3, esq2):
    m = flat.shape[0]
    n_blocks = m // M_TILE
    return pl.pallas_call(
        _dist_argmin_body,
        grid=(n_blocks,),
        in_specs=[
            pl.BlockSpec((M_TILE, DIM), lambda i: (i, 0)),
            pl.BlockSpec((KSIZE, DIM), lambda i: (0, 0)),
            pl.BlockSpec((1, 1, M_TILE), lambda i: (i, 0, 0)),
            pl.BlockSpec((1, KSIZE), lambda i: (0, 0)),
        ],
        out_specs=[
            pl.BlockSpec((M_TILE,), lambda i: (i,)),
            pl.BlockSpec(memory_space=pltpu.SMEM),
        ],
        out_shape=[
            jax.ShapeDtypeStruct((m,), jnp.int32),
            jax.ShapeDtypeStruct((1, 1), jnp.float32),
        ],
        scratch_shapes=[pltpu.SMEM((1,), jnp.float32)],
    )(flat, cbt, xsq3, esq2)


def _sc_gather(codebook, idx_flat):
    nc, ns = 2, 16
    nw = nc * ns
    b = idx_flat.shape[0]
    b_per_w = b // nw
    mesh = plsc.VectorSubcoreMesh(core_axis_name="c", subcore_axis_name="s")

    @functools.partial(
        pl.kernel,
        mesh=mesh,
        out_type=jax.ShapeDtypeStruct((b, DIM), jnp.float32),
        scratch_types=[
            pltpu.VMEM((b_per_w,), jnp.int32),
            pltpu.VMEM((b_per_w, DIM), jnp.float32),
            pltpu.SemaphoreType.DMA,
        ],
    )
    def gather_kernel(cb_hbm, idx_hbm, out_hbm, idx_v, rows_v, sem):
        wid = lax.axis_index("s") * nc + lax.axis_index("c")
        base = wid * b_per_w
        pltpu.sync_copy(idx_hbm.at[pl.ds(base, b_per_w)], idx_v)
        pltpu.async_copy(cb_hbm.at[idx_v], rows_v, sem).wait()
        pltpu.sync_copy(rows_v, out_hbm.at[pl.ds(base, b_per_w)])

    return gather_kernel(codebook, idx_flat)


def kernel(x, codebook):
    b, n, d = x.shape
    flat = x.reshape(-1, d)
    m = flat.shape[0]
    x_sq = jnp.sum(flat * flat, axis=-1, keepdims=True)
    e_sq = jnp.sum(codebook * codebook, axis=-1)

    xsq3 = x_sq.reshape(m // M_TILE, 1, M_TILE)
    esq2 = e_sq.reshape(1, KSIZE)

    idx_flat, loss = _dist_argmin(flat, codebook, xsq3, esq2)

    quantized = _sc_gather(codebook, idx_flat)

    return quantized.reshape(b, n, d), idx_flat.reshape(b, n), loss[0, 0]


# speedup vs baseline: 1.1245x; 1.0889x over previous
import functools

import jax
import jax.numpy as jnp
from jax import lax
from jax.experimental import pallas as pl
from jax.experimental.pallas import tpu as pltpu
from jax.experimental.pallas import tpu_sc as plsc

DIM = 256
KSIZE = 8192
M_TILE = 512
K_WIN = 2048


def _bf16_round(v):
    return v.astype(jnp.bfloat16).astype(jnp.float32)


def _dist_argmin_body(x_ref, cbt_ref, xsq_ref, esq_ref, idx_ref, loss_ref,
                      acc_ref):
    i = pl.program_id(0)

    @pl.when(i == 0)
    def _():
        acc_ref[0] = 0.0

    x = x_ref[...]
    xsq = xsq_ref[0, 0, :][:, None]

    iota_l = lax.broadcasted_iota(jnp.int32, (M_TILE, K_WIN), 1)
    acc_v = acc_i = fresh_v = None
    for w in range(KSIZE // K_WIN):
        off = w * K_WIN
        cb_w = cbt_ref[pl.ds(off, K_WIN), :]
        dots = lax.dot_general(x, cb_w, (((1,), (1,)), ((), ())),
                               preferred_element_type=jnp.float32)
        esq_w = esq_ref[:, pl.ds(off, K_WIN)]
        dist = xsq - 2.0 * dots + esq_w
        wv = jnp.min(dist, axis=1)
        wi = jnp.min(jnp.where(dist == wv[:, None], iota_l, KSIZE),
                     axis=1) + off
        if acc_v is None:
            acc_v, acc_i, fresh_v = _bf16_round(wv), wi, wv
        else:
            upd = wv < acc_v
            acc_i = jnp.where(upd, wi, acc_i)
            fresh_v = jnp.where(upd, wv, fresh_v)
            acc_v = jnp.where(upd, _bf16_round(wv), acc_v)

    idx_ref[...] = acc_i
    acc_ref[0] += jnp.sum(fresh_v)

    @pl.when(i == pl.num_programs(0) - 1)
    def _():
        loss_ref[0, 0] = acc_ref[0] * (1.0 / (KSIZE * DIM))


def _dist_argmin(flat, cbt, xsq3, esq2):
    m = flat.shape[0]
    n_blocks = m // M_TILE
    return pl.pallas_call(
        _dist_argmin_body,
        grid=(n_blocks,),
        in_specs=[
            pl.BlockSpec((M_TILE, DIM), lambda i: (i, 0)),
            pl.BlockSpec((KSIZE, DIM), lambda i: (0, 0)),
            pl.BlockSpec((1, 1, M_TILE), lambda i: (i, 0, 0)),
            pl.BlockSpec((1, KSIZE), lambda i: (0, 0)),
        ],
        out_specs=[
            pl.BlockSpec((M_TILE,), lambda i: (i,)),
            pl.BlockSpec(memory_space=pltpu.SMEM),
        ],
        out_shape=[
            jax.ShapeDtypeStruct((m,), jnp.int32),
            jax.ShapeDtypeStruct((1, 1), jnp.float32),
        ],
        scratch_shapes=[pltpu.SMEM((1,), jnp.float32)],
    )(flat, cbt, xsq3, esq2)


def _sc_gather(codebook, idx_flat):
    nc, ns = 2, 16
    nw = nc * ns
    b = idx_flat.shape[0]
    b_per_w = b // nw
    mesh = plsc.VectorSubcoreMesh(core_axis_name="c", subcore_axis_name="s")

    @functools.partial(
        pl.kernel,
        mesh=mesh,
        out_type=jax.ShapeDtypeStruct((b, DIM), jnp.float32),
        scratch_types=[
            pltpu.VMEM((b_per_w,), jnp.int32),
            pltpu.VMEM((b_per_w, DIM), jnp.float32),
            pltpu.SemaphoreType.DMA,
        ],
    )
    def gather_kernel(cb_hbm, idx_hbm, out_hbm, idx_v, rows_v, sem):
        wid = lax.axis_index("s") * nc + lax.axis_index("c")
        base = wid * b_per_w
        pltpu.sync_copy(idx_hbm.at[pl.ds(base, b_per_w)], idx_v)
        pltpu.async_copy(cb_hbm.at[idx_v], rows_v, sem).wait()
        pltpu.sync_copy(rows_v, out_hbm.at[pl.ds(base, b_per_w)])

    return gather_kernel(codebook, idx_flat)


def kernel(x, codebook):
    b, n, d = x.shape
    flat = x.reshape(-1, d)
    m = flat.shape[0]
    x_sq = jnp.sum(flat * flat, axis=-1, keepdims=True)
    e_sq = jnp.sum(codebook * codebook, axis=-1)

    xsq3 = x_sq.reshape(m // M_TILE, 1, M_TILE)
    esq2 = e_sq.reshape(1, KSIZE)

    idx_flat, loss = _dist_argmin(flat, codebook, xsq3, esq2)

    quantized = _sc_gather(codebook, idx_flat)

    return quantized.reshape(b, n, d), idx_flat.reshape(b, n), loss[0, 0]


# speedup vs baseline: 1.1991x; 1.0663x over previous
import functools

import jax
import jax.numpy as jnp
from jax import lax
from jax.experimental import pallas as pl
from jax.experimental.pallas import tpu as pltpu
from jax.experimental.pallas import tpu_sc as plsc

DIM = 256
KSIZE = 8192
M_TILE = 1024
K_WIN = 2048


def _bf16_round(v):
    return v.astype(jnp.bfloat16).astype(jnp.float32)


def _dist_argmin_body(x_ref, cbt_ref, xsq_ref, esq_ref, idx_ref, loss_ref,
                      acc_ref):
    i = pl.program_id(0)

    @pl.when(i == 0)
    def _():
        acc_ref[0] = 0.0

    x = x_ref[...]
    xsq = xsq_ref[0, 0, :][:, None]

    iota_l = lax.broadcasted_iota(jnp.int32, (M_TILE, K_WIN), 1)
    acc_v = acc_i = fresh_v = None
    for w in range(KSIZE // K_WIN):
        off = w * K_WIN
        cb_w = cbt_ref[pl.ds(off, K_WIN), :]
        dots = lax.dot_general(x, cb_w, (((1,), (1,)), ((), ())),
                               preferred_element_type=jnp.float32)
        esq_w = esq_ref[:, pl.ds(off, K_WIN)]
        dist = xsq - 2.0 * dots + esq_w
        wv = jnp.min(dist, axis=1)
        wi = jnp.min(jnp.where(dist == wv[:, None], iota_l, KSIZE),
                     axis=1) + off
        if acc_v is None:
            acc_v, acc_i, fresh_v = _bf16_round(wv), wi, wv
        else:
            upd = wv < acc_v
            acc_i = jnp.where(upd, wi, acc_i)
            fresh_v = jnp.where(upd, wv, fresh_v)
            acc_v = jnp.where(upd, _bf16_round(wv), acc_v)

    idx_ref[...] = acc_i
    acc_ref[0] += jnp.sum(fresh_v)

    @pl.when(i == pl.num_programs(0) - 1)
    def _():
        loss_ref[0, 0] = acc_ref[0] * (1.0 / (KSIZE * DIM))


def _dist_argmin(flat, cbt, xsq3, esq2):
    m = flat.shape[0]
    n_blocks = m // M_TILE
    return pl.pallas_call(
        _dist_argmin_body,
        grid=(n_blocks,),
        in_specs=[
            pl.BlockSpec((M_TILE, DIM), lambda i: (i, 0)),
            pl.BlockSpec((KSIZE, DIM), lambda i: (0, 0)),
            pl.BlockSpec((1, 1, M_TILE), lambda i: (i, 0, 0)),
            pl.BlockSpec((1, KSIZE), lambda i: (0, 0)),
        ],
        out_specs=[
            pl.BlockSpec((M_TILE,), lambda i: (i,)),
            pl.BlockSpec(memory_space=pltpu.SMEM),
        ],
        out_shape=[
            jax.ShapeDtypeStruct((m,), jnp.int32),
            jax.ShapeDtypeStruct((1, 1), jnp.float32),
        ],
        scratch_shapes=[pltpu.SMEM((1,), jnp.float32)],
    )(flat, cbt, xsq3, esq2)


def _sc_gather(codebook, idx_flat):
    nc, ns = 2, 16
    nw = nc * ns
    b = idx_flat.shape[0]
    b_per_w = b // nw
    mesh = plsc.VectorSubcoreMesh(core_axis_name="c", subcore_axis_name="s")

    @functools.partial(
        pl.kernel,
        mesh=mesh,
        out_type=jax.ShapeDtypeStruct((b, DIM), jnp.float32),
        scratch_types=[
            pltpu.VMEM((b_per_w,), jnp.int32),
            pltpu.VMEM((b_per_w, DIM), jnp.float32),
            pltpu.SemaphoreType.DMA,
        ],
    )
    def gather_kernel(cb_hbm, idx_hbm, out_hbm, idx_v, rows_v, sem):
        wid = lax.axis_index("s") * nc + lax.axis_index("c")
        base = wid * b_per_w
        pltpu.sync_copy(idx_hbm.at[pl.ds(base, b_per_w)], idx_v)
        pltpu.async_copy(cb_hbm.at[idx_v], rows_v, sem).wait()
        pltpu.sync_copy(rows_v, out_hbm.at[pl.ds(base, b_per_w)])

    return gather_kernel(codebook, idx_flat)


def kernel(x, codebook):
    b, n, d = x.shape
    flat = x.reshape(-1, d)
    m = flat.shape[0]
    x_sq = jnp.sum(flat * flat, axis=-1, keepdims=True)
    e_sq = jnp.sum(codebook * codebook, axis=-1)

    xsq3 = x_sq.reshape(m // M_TILE, 1, M_TILE)
    esq2 = e_sq.reshape(1, KSIZE)

    idx_flat, loss = _dist_argmin(flat, codebook, xsq3, esq2)

    quantized = _sc_gather(codebook, idx_flat)

    return quantized.reshape(b, n, d), idx_flat.reshape(b, n), loss[0, 0]


# speedup vs baseline: 1.3028x; 1.0865x over previous
import functools

import jax
import jax.numpy as jnp
from jax import lax
from jax.experimental import pallas as pl
from jax.experimental.pallas import tpu as pltpu
from jax.experimental.pallas import tpu_sc as plsc

DIM = 256
KSIZE = 8192
M_TILE = 1024
K_WIN = 2048


def _bf16_round(v):
    return v.astype(jnp.bfloat16).astype(jnp.float32)


def _dist_argmin_body(x_ref, cbt_ref, xsq_ref, esq_ref, idx_ref, loss_ref,
                      acc_ref):
    i = pl.program_id(0)

    @pl.when(i == 0)
    def _():
        acc_ref[0] = 0.0

    x = x_ref[...]
    xsq = xsq_ref[0, 0, :][:, None]

    iota_l = lax.broadcasted_iota(
        jnp.int32, (M_TILE, K_WIN), 1).astype(jnp.float32)
    acc_v = acc_i = fresh_v = None
    for w in range(KSIZE // K_WIN):
        off = w * K_WIN
        cb_w = cbt_ref[pl.ds(off, K_WIN), :]
        dots = lax.dot_general(x, cb_w, (((1,), (1,)), ((), ())),
                               preferred_element_type=jnp.float32)
        esq_w = esq_ref[:, pl.ds(off, K_WIN)]
        dist = (xsq + dots) + esq_w
        wv = jnp.min(dist, axis=1)
        wi = jnp.min(jnp.where(dist == wv[:, None], iota_l, float(KSIZE)),
                     axis=1).astype(jnp.int32) + off
        if acc_v is None:
            acc_v, acc_i, fresh_v = _bf16_round(wv), wi, wv
        else:
            upd = wv < acc_v
            acc_i = jnp.where(upd, wi, acc_i)
            fresh_v = jnp.where(upd, wv, fresh_v)
            acc_v = jnp.where(upd, _bf16_round(wv), acc_v)

    idx_ref[...] = acc_i
    acc_ref[0] += jnp.sum(fresh_v)

    @pl.when(i == pl.num_programs(0) - 1)
    def _():
        loss_ref[0, 0] = acc_ref[0] * (1.0 / (KSIZE * DIM))


def _dist_argmin(flat, cbt, xsq3, esq2):
    m = flat.shape[0]
    n_blocks = m // M_TILE
    return pl.pallas_call(
        _dist_argmin_body,
        grid=(n_blocks,),
        in_specs=[
            pl.BlockSpec((M_TILE, DIM), lambda i: (i, 0)),
            pl.BlockSpec((KSIZE, DIM), lambda i: (0, 0)),
            pl.BlockSpec((1, 1, M_TILE), lambda i: (i, 0, 0)),
            pl.BlockSpec((1, KSIZE), lambda i: (0, 0)),
        ],
        out_specs=[
            pl.BlockSpec((M_TILE,), lambda i: (i,)),
            pl.BlockSpec(memory_space=pltpu.SMEM),
        ],
        out_shape=[
            jax.ShapeDtypeStruct((m,), jnp.int32),
            jax.ShapeDtypeStruct((1, 1), jnp.float32),
        ],
        scratch_shapes=[pltpu.SMEM((1,), jnp.float32)],
    )(flat, cbt, xsq3, esq2)


def _sc_gather(codebook, idx_flat):
    nc, ns = 2, 16
    nw = nc * ns
    b = idx_flat.shape[0]
    b_per_w = b // nw
    mesh = plsc.VectorSubcoreMesh(core_axis_name="c", subcore_axis_name="s")

    @functools.partial(
        pl.kernel,
        mesh=mesh,
        out_type=jax.ShapeDtypeStruct((b, DIM), jnp.float32),
        scratch_types=[
            pltpu.VMEM((b_per_w,), jnp.int32),
            pltpu.VMEM((b_per_w, DIM), jnp.float32),
            pltpu.SemaphoreType.DMA,
        ],
    )
    def gather_kernel(cb_hbm, idx_hbm, out_hbm, idx_v, rows_v, sem):
        wid = lax.axis_index("s") * nc + lax.axis_index("c")
        base = wid * b_per_w
        pltpu.sync_copy(idx_hbm.at[pl.ds(base, b_per_w)], idx_v)
        pltpu.async_copy(cb_hbm.at[idx_v], rows_v, sem).wait()
        pltpu.sync_copy(rows_v, out_hbm.at[pl.ds(base, b_per_w)])

    return gather_kernel(codebook, idx_flat)


def kernel(x, codebook):
    b, n, d = x.shape
    flat = x.reshape(-1, d)
    m = flat.shape[0]
    x_sq = jnp.sum(flat * flat, axis=-1, keepdims=True)
    e_sq = jnp.sum(codebook * codebook, axis=-1)
    flatn = -2.0 * flat

    xsq3 = x_sq.reshape(m // M_TILE, 1, M_TILE)
    esq2 = e_sq.reshape(1, KSIZE)

    idx_flat, loss = _dist_argmin(flatn, codebook, xsq3, esq2)

    quantized = _sc_gather(codebook, idx_flat)

    return quantized.reshape(b, n, d), idx_flat.reshape(b, n), loss[0, 0]
